# Initial kernel scaffold; baseline (speedup 1.0000x reference)
import jax
import jax.numpy as jnp
from jax.experimental import pallas as pl


def kernel(embeddings, labels):
    raise NotImplementedError("write your pallas kernel here")


# speedup vs baseline: 3.8313x; 3.8313x over previous
import functools

import jax
import jax.numpy as jnp
from jax import lax
from jax.experimental import pallas as pl
from jax.experimental.pallas import tpu as pltpu
from jax.experimental.pallas import tpu_sc as plsc

N = 16384
D = 64
KPAD = 128
NC = 2
NS = 16
NW = NC * NS
CHUNK = N // NW
NB = CHUNK // 128
W = D + 16


def _sc_body(emb_hbm, lab_hbm, out_sums, out_sqa,
             lab_v, emb_v, sqa_v, zb64, zb80, sh_sums, sh_sqa):
  c = lax.axis_index("c")
  s = lax.axis_index("s")
  wid = s * NC + c

  zv = jnp.zeros((16,), jnp.float32)
  ones_v = jnp.ones((16,), jnp.float32)

  for i in range(KPAD // NS):
    for j in range(D // 16):
      zb64[i, pl.ds(j * 16, 16)] = zv
    for j in range(W // 16):
      zb80[i, pl.ds(j * 16, 16)] = zv
  rows = KPAD // NS
  pltpu.sync_copy(zb64, sh_sums.at[pl.ds(s * rows, rows)])
  pltpu.sync_copy(zb80, sh_sqa.at[pl.ds(s * rows, rows)])

  pltpu.sync_copy(emb_hbm.at[pl.ds(wid * CHUNK, CHUNK)], emb_v)
  pltpu.sync_copy(lab_hbm.at[pl.ds(wid * NB, NB)], lab_v)

  plsc.subcore_barrier()

  for b in range(NB):
    def row_body(i, carry):
      for j in range(D // 16):
        v = emb_v[b * 128 + i, pl.ds(j * 16, 16)]
        sqa_v[i, pl.ds(j * 16, 16)] = v * v
      sqa_v[i, pl.ds(D, 16)] = ones_v
      return carry

    lax.fori_loop(0, 128, row_body, 0)
    pltpu.sync_copy(emb_v.at[pl.ds(b * 128, 128)],
                    sh_sums.at[lab_v.at[b]], add=True)
    pltpu.sync_copy(sqa_v, sh_sqa.at[lab_v.at[b]], add=True)

  plsc.subcore_barrier()

  @pl.when(s == 0)
  def _dump():
    pltpu.sync_copy(sh_sums, out_sums.at[c])
    pltpu.sync_copy(sh_sqa, out_sqa.at[c])


@functools.partial(
    pl.kernel,
    out_type=(
        jax.ShapeDtypeStruct((NC, KPAD, D), jnp.float32),
        jax.ShapeDtypeStruct((NC, KPAD, W), jnp.float32),
    ),
    mesh=plsc.VectorSubcoreMesh(
        core_axis_name="c", subcore_axis_name="s",
        num_cores=NC, num_subcores=NS),
    compiler_params=pltpu.CompilerParams(use_tc_tiling_on_sc=False),
    scratch_types=[
        pltpu.VMEM((NB, 128), jnp.int32),
        pltpu.VMEM((CHUNK, D), jnp.float32),
        pltpu.VMEM((128, W), jnp.float32),
        pltpu.VMEM((KPAD // NS, D), jnp.float32),
        pltpu.VMEM((KPAD // NS, W), jnp.float32),
        pltpu.VMEM_SHARED((KPAD, D), jnp.float32),
        pltpu.VMEM_SHARED((KPAD, W), jnp.float32),
    ],
)
def _sc_partials(emb_hbm, lab_hbm, out_sums, out_sqa, *scratch):
  _sc_body(emb_hbm, lab_hbm, out_sums, out_sqa, *scratch)


def _tc_combine_body(sums_ref, sqa_ref, out_ref):
  sums = sums_ref[0] + sums_ref[1]
  sqa = sqa_ref[0] + sqa_ref[1]
  sq = sqa[:, :D]
  cnt = sqa[:, D:D + 1]
  sumsq = jnp.sum(sq, axis=1, keepdims=True)
  normsq = jnp.sum(sums * sums, axis=1, keepdims=True)
  safe = jnp.maximum(cnt, 1.0)
  per_class = (sumsq - normsq / safe) / safe
  present = cnt > 0.0
  n_unique = jnp.sum(present.astype(jnp.float32))
  out_ref[0, 0] = jnp.sum(jnp.where(present, per_class, 0.0)) / n_unique


def kernel(embeddings, labels):
  lab = labels.astype(jnp.int32).reshape(NW * NB, 128)
  sums_p, sqa_p = _sc_partials(embeddings, lab)
  loss = pl.pallas_call(
      _tc_combine_body,
      out_shape=jax.ShapeDtypeStruct((1, 1), jnp.float32),
      out_specs=pl.BlockSpec(memory_space=pltpu.SMEM),
  )(sums_p, sqa_p)
  return loss[0, 0]
